# baseline (speedup 1.0000x reference)
import functools
import math

import jax
import jax.numpy as jnp
from jax.experimental import pallas as pl

PARTITIONS = 8
D = 128
TWO_PI = 2.0 * math.pi
HALF_PI = 0.5 * math.pi
PI = math.pi

_ATAN_C0 = 0.99998007
_ATAN_C1 = -0.33269451
_ATAN_C2 = 0.19402051
_ATAN_C3 = -0.11769694
_ATAN_C4 = 0.05408478
_ATAN_C5 = -0.01230061


def _atan2_fast(a, b):
    ax = jnp.abs(a)
    ab = jnp.abs(b)
    mx = jnp.maximum(ax, ab)
    mn = jnp.minimum(ax, ab)
    t = mn / jnp.maximum(mx, 1e-37)
    s = t * t
    p = _ATAN_C5
    p = p * s + _ATAN_C4
    p = p * s + _ATAN_C3
    p = p * s + _ATAN_C2
    p = p * s + _ATAN_C1
    p = p * s + _ATAN_C0
    q = p * t
    q = jnp.where(ax > ab, HALF_PI - q, q)
    q = jnp.where(b < 0, PI - q, q)
    return jnp.where(a < 0, -q, q)


def _circle_kernel(nei_ref, res_ref, ego_ref, w_ref, b_ref, out_ref):
    nei = nei_ref[...]
    res = res_ref[...]
    Bb, _, N = nei.shape

    traj_sum = jnp.sum(nei, axis=1)
    mask = traj_sum != 0

    relx = nei[:, 38, :] - ego_ref[:, :, 0]
    rely = nei[:, 39, :] - ego_ref[:, :, 1]
    d2 = relx * relx + rely * rely
    f_distance = d2 * jax.lax.rsqrt(jnp.maximum(d2, 1e-37))
    f_direction = _atan2_fast(relx, rely)
    f_direction = jnp.where(f_direction < 0, f_direction + TWO_PI, f_direction)
    idx = (f_direction * (PARTITIONS / TWO_PI)).astype(jnp.int32)
    idx = jnp.where(mask, idx, -1)

    piota = jax.lax.broadcasted_iota(jnp.int32, (Bb, PARTITIONS, N), 1)
    oh = (idx[:, None, :] == piota).astype(jnp.float32)

    scan2 = jnp.concatenate(
        [f_distance[:, None, :], f_direction[:, None, :]], axis=1)

    res_sums = jax.lax.dot_general(
        oh, res, (((2,), (2,)), ((0,), (0,))),
        preferred_element_type=jnp.float32)
    scan_sums = jax.lax.dot_general(
        oh, scan2, (((2,), (2,)), ((0,), (0,))),
        preferred_element_type=jnp.float32)

    ones64 = jnp.ones((64, N), jnp.float32)
    cnt_rep = jax.lax.dot_general(
        oh, ones64, (((2,), (1,)), ((), ())),
        preferred_element_type=jnp.float32)
    inv_rep = 1.0 / (cnt_rep + 0.0001)

    w = w_ref[...]
    b = b_ref[...]
    h = jax.lax.dot_general(
        scan_sums, w, (((2,), (0,)), ((), ())),
        preferred_element_type=jnp.float32)
    f_scan = jax.nn.relu(h * inv_rep + b[0][None, None, :])

    out_ref[:, :, :64] = res_sums * inv_rep
    out_ref[:, :, 64:] = f_scan


@functools.partial(jax.jit, static_argnames=("interpret", "bb"))
def kernel(ego_traj_2d, nei_traj_2d, f_resonance, W, b, interpret=False, bb=128):
    B, N, T, _ = nei_traj_2d.shape
    nei_T = nei_traj_2d.transpose(0, 2, 3, 1).reshape(B, T * 2, N)
    ego_last = ego_traj_2d[:, -1, :].reshape(B, 1, 2)
    res_T = f_resonance.transpose(0, 2, 1)
    b2 = b.reshape(1, -1)

    grid = (B // bb,)
    out = pl.pallas_call(
        _circle_kernel,
        grid=grid,
        in_specs=[
            pl.BlockSpec((bb, T * 2, N), lambda i: (i, 0, 0)),
            pl.BlockSpec((bb, 64, N), lambda i: (i, 0, 0)),
            pl.BlockSpec((bb, 1, 2), lambda i: (i, 0, 0)),
            pl.BlockSpec((2, 64), lambda i: (0, 0)),
            pl.BlockSpec((1, 64), lambda i: (0, 0)),
        ],
        out_specs=pl.BlockSpec((bb, PARTITIONS, D), lambda i: (i, 0, 0)),
        out_shape=jax.ShapeDtypeStruct((B, PARTITIONS, D), jnp.float32),
        interpret=interpret,
    )(nei_T, res_T, ego_last, W, b2)
    return out


# speedup vs baseline: 1.1199x; 1.1199x over previous
import functools
import math

import jax
import jax.numpy as jnp
from jax.experimental import pallas as pl

PARTITIONS = 8
D = 128
TWO_PI = 2.0 * math.pi
HALF_PI = 0.5 * math.pi
PI = math.pi

_ATAN_C0 = 0.99998007
_ATAN_C1 = -0.33269451
_ATAN_C2 = 0.19402051
_ATAN_C3 = -0.11769694
_ATAN_C4 = 0.05408478
_ATAN_C5 = -0.01230061


def _atan2_fast(a, b):
    ax = jnp.abs(a)
    ab = jnp.abs(b)
    mx = jnp.maximum(ax, ab)
    mn = jnp.minimum(ax, ab)
    t = mn / jnp.maximum(mx, 1e-37)
    s = t * t
    p = _ATAN_C5
    p = p * s + _ATAN_C4
    p = p * s + _ATAN_C3
    p = p * s + _ATAN_C2
    p = p * s + _ATAN_C1
    p = p * s + _ATAN_C0
    q = p * t
    q = jnp.where(ax > ab, HALF_PI - q, q)
    q = jnp.where(b < 0, PI - q, q)
    return jnp.where(a < 0, -q, q)


def _circle_kernel(nei_ref, res_ref, ego_ref, w_ref, b_ref, out_ref):
    nei = nei_ref[...]
    res = res_ref[...]
    Bb, _, N = nei.shape

    traj_sum = jnp.sum(nei, axis=1)
    mask = traj_sum != 0

    i = pl.program_id(0)
    Bbs = nei.shape[0]
    ego = ego_ref[pl.ds(i * Bbs, Bbs), :, :]
    relx = nei[:, 38, :] - ego[:, :, 0]
    rely = nei[:, 39, :] - ego[:, :, 1]
    d2 = relx * relx + rely * rely
    f_distance = d2 * jax.lax.rsqrt(jnp.maximum(d2, 1e-37))
    f_direction = _atan2_fast(relx, rely)
    f_direction = jnp.where(f_direction < 0, f_direction + TWO_PI, f_direction)
    idx = (f_direction * (PARTITIONS / TWO_PI)).astype(jnp.int32)
    idx = jnp.where(mask, idx, -1)

    piota = jax.lax.broadcasted_iota(jnp.int32, (Bb, PARTITIONS, N), 1)
    oh = (idx[:, None, :] == piota).astype(jnp.float32)

    scan2 = jnp.concatenate(
        [f_distance[:, None, :], f_direction[:, None, :]], axis=1)

    res_sums = jax.lax.dot_general(
        oh, res, (((2,), (2,)), ((0,), (0,))),
        preferred_element_type=jnp.float32)
    scan_sums = jax.lax.dot_general(
        oh, scan2, (((2,), (2,)), ((0,), (0,))),
        preferred_element_type=jnp.float32)

    ones64 = jnp.ones((64, N), jnp.float32)
    cnt_rep = jax.lax.dot_general(
        oh, ones64, (((2,), (1,)), ((), ())),
        preferred_element_type=jnp.float32)
    inv_rep = 1.0 / (cnt_rep + 0.0001)

    w = w_ref[...]
    b = b_ref[...]
    h = jax.lax.dot_general(
        scan_sums, w, (((2,), (0,)), ((), ())),
        preferred_element_type=jnp.float32)
    f_scan = jax.nn.relu(h * inv_rep + b[0][None, None, :])

    out_ref[:, :, :64] = res_sums * inv_rep
    out_ref[:, :, 64:] = f_scan


@functools.partial(jax.jit, static_argnames=("interpret", "bb"))
def kernel(ego_traj_2d, nei_traj_2d, f_resonance, W, b, interpret=False, bb=256):
    B, N, T, _ = nei_traj_2d.shape
    nei_T = nei_traj_2d.transpose(0, 2, 3, 1).reshape(B, T * 2, N)
    ego_last = ego_traj_2d[:, -1, :].reshape(B, 1, 2)
    res_T = f_resonance.transpose(0, 2, 1)
    b2 = b.reshape(1, -1)

    grid = (B // bb,)
    out = pl.pallas_call(
        _circle_kernel,
        grid=grid,
        in_specs=[
            pl.BlockSpec((bb, T * 2, N), lambda i: (i, 0, 0)),
            pl.BlockSpec((bb, 64, N), lambda i: (i, 0, 0)),
            pl.BlockSpec((B, 1, 2), lambda i: (0, 0, 0)),
            pl.BlockSpec((2, 64), lambda i: (0, 0)),
            pl.BlockSpec((1, 64), lambda i: (0, 0)),
        ],
        out_specs=pl.BlockSpec((bb, PARTITIONS, D), lambda i: (i, 0, 0)),
        out_shape=jax.ShapeDtypeStruct((B, PARTITIONS, D), jnp.float32),
        interpret=interpret,
    )(nei_T, res_T, ego_last, W, b2)
    return out
